# Initial kernel scaffold; baseline (speedup 1.0000x reference)
import jax
import jax.numpy as jnp
from jax.experimental import pallas as pl


def kernel(node_features, edge_src, edge_dst, edge_attr, edge_scalars, W1, W2):
    raise NotImplementedError("write your pallas kernel here")


# speedup vs baseline: 1.2944x; 1.2944x over previous
import functools
import math

import jax
import jax.numpy as jnp
from jax import lax
from jax.experimental import pallas as pl
from jax.experimental.pallas import tpu as pltpu
from jax.experimental.pallas import tpu_sc as plsc

N = 10000
E = 160000
D = 16
HID = 256

NC = 2
NS = 16
NW = NC * NS
CH = 128
NCH = 40
PER_W = NCH * CH
E_PAD = NW * PER_W
ZR = N // NS


def _sc_gather_body(nf_hbm, src_hbm, out_hbm, idx_v, rows_v, sem):
    c = lax.axis_index("c")
    s = lax.axis_index("s")
    wid = s * NC + c
    pltpu.sync_copy(src_hbm.at[wid], idx_v)
    for g in range(0, NCH, 8):
        cps = [
            pltpu.async_copy(nf_hbm.at[idx_v.at[g + b]], rows_v.at[g + b], sem)
            for b in range(8)
        ]
        for cp in cps:
            cp.wait()
    pltpu.sync_copy(rows_v, out_hbm.at[wid])


def _sc_scatter_body(ef_hbm, dst_hbm, zero_hbm, part_hbm, idx_v, ef_v, acc, sem):
    c = lax.axis_index("c")
    s = lax.axis_index("s")
    wid = s * NC + c
    pltpu.sync_copy(zero_hbm.at[pl.ds(s * ZR, ZR)], acc.at[pl.ds(s * ZR, ZR)])
    pltpu.sync_copy(dst_hbm.at[wid], idx_v)
    pltpu.sync_copy(ef_hbm.at[wid], ef_v)
    plsc.subcore_barrier()
    for g in range(0, NCH, 8):
        cps = [
            pltpu.async_copy(ef_v.at[g + b], acc.at[idx_v.at[g + b]], sem, add=True)
            for b in range(8)
        ]
        for cp in cps:
            cp.wait()
    plsc.subcore_barrier()
    pltpu.sync_copy(acc.at[pl.ds(s * ZR, ZR)], part_hbm.at[c, pl.ds(s * ZR, ZR)])


_BLK = 2048
_INV_SQRT3 = 1.0 / math.sqrt(3.0)


def _tc_edge_body(scal_ref, attr_ref, x_ref, w1_ref, w2_ref, out_ref):
    h = jnp.dot(scal_ref[...], w1_ref[...], preferred_element_type=jnp.float32)
    h = jnp.maximum(h * _INV_SQRT3, 0.0)
    w = jnp.dot(h, w2_ref[...], preferred_element_type=jnp.float32) * (1.0 / 16.0)
    x = x_ref[...]
    acc = x[:, 0:1] * w[:, 0:D]
    for i in range(1, D):
        acc = acc + x[:, i : i + 1] * w[:, i * D : (i + 1) * D]
    out_ref[...] = acc * attr_ref[...] * (1.0 / 16.0)


def _tc_edge(scal, attr, x, w1, w2):
    return pl.pallas_call(
        _tc_edge_body,
        grid=(E_PAD // _BLK,),
        in_specs=[
            pl.BlockSpec((_BLK, 3), lambda i: (i, 0)),
            pl.BlockSpec((_BLK, 1), lambda i: (i, 0)),
            pl.BlockSpec((_BLK, D), lambda i: (i, 0)),
            pl.BlockSpec((3, HID), lambda i: (0, 0)),
            pl.BlockSpec((HID, HID), lambda i: (0, 0)),
        ],
        out_specs=pl.BlockSpec((_BLK, D), lambda i: (i, 0)),
        out_shape=jax.ShapeDtypeStruct((E_PAD, D), jnp.float32),
    )(scal, attr, x, w1, w2)


def _tc_combine_body(p_ref, o_ref):
    o_ref[...] = p_ref[0] + p_ref[1]


def _tc_combine(parts):
    return pl.pallas_call(
        _tc_combine_body,
        grid=(5,),
        in_specs=[pl.BlockSpec((NC, N // 5, D), lambda i: (0, i, 0))],
        out_specs=pl.BlockSpec((N // 5, D), lambda i: (i, 0)),
        out_shape=jax.ShapeDtypeStruct((N, D), jnp.float32),
    )(parts)


@functools.cache
def _sc_kernels():
    mesh = plsc.VectorSubcoreMesh(core_axis_name="c", subcore_axis_name="s")
    gather = pl.kernel(
        _sc_gather_body,
        out_type=jax.ShapeDtypeStruct((NW, NCH, CH, D), jnp.float32),
        mesh=mesh,
        scratch_types=[
            pltpu.VMEM((NCH, CH), jnp.int32),
            pltpu.VMEM((NCH, CH, D), jnp.float32),
            pltpu.SemaphoreType.DMA,
        ],
        compiler_params=pltpu.CompilerParams(use_tc_tiling_on_sc=False),
    )
    scatter = pl.kernel(
        _sc_scatter_body,
        out_type=jax.ShapeDtypeStruct((NC, N, D), jnp.float32),
        mesh=mesh,
        scratch_types=[
            pltpu.VMEM((NCH, CH), jnp.int32),
            pltpu.VMEM((NCH, CH, D), jnp.float32),
            pltpu.VMEM_SHARED((N, D), jnp.float32),
            pltpu.SemaphoreType.DMA,
        ],
        compiler_params=pltpu.CompilerParams(use_tc_tiling_on_sc=False),
    )
    return gather, scatter


def kernel(node_features, edge_src, edge_dst, edge_attr, edge_scalars, W1, W2):
    _sc_gather, _sc_scatter = _sc_kernels()
    pad = E_PAD - E
    src = jnp.pad(edge_src, (0, pad)).reshape(NW, NCH, CH)
    dst = jnp.pad(edge_dst, (0, pad)).reshape(NW, NCH, CH)
    attr = jnp.pad(edge_attr, ((0, pad), (0, 0)))
    scal = jnp.pad(edge_scalars, ((0, pad), (0, 0)))

    x = _sc_gather(node_features, src).reshape(E_PAD, D)
    ef = _tc_edge(scal, attr, x, W1, W2).reshape(NW, NCH, CH, D)
    zeros = jnp.zeros((N, D), jnp.float32)
    parts = _sc_scatter(ef, dst, zeros)
    return _tc_combine(parts)


# speedup vs baseline: 2.7873x; 2.1534x over previous
import functools
import math

import jax
import jax.numpy as jnp
from jax import lax
from jax.experimental import pallas as pl
from jax.experimental.pallas import tpu as pltpu
from jax.experimental.pallas import tpu_sc as plsc

N = 10000
E = 160000
D = 16
HID = 256

NC = 2
NS = 16
NW = NC * NS
CH = 128
NCH = 40
PER_W = NCH * CH
E_PAD = NW * PER_W
ZR = N // NS


def _sc_gather_body(nf_hbm, src_hbm, out_hbm, idx_v, rows_v, sem):
    c = lax.axis_index("c")
    s = lax.axis_index("s")
    wid = s * NC + c
    pltpu.sync_copy(src_hbm.at[wid], idx_v)
    for g in range(0, NCH, 8):
        cps = [
            pltpu.async_copy(nf_hbm.at[idx_v.at[g + b]], rows_v.at[g + b], sem)
            for b in range(8)
        ]
        for cp in cps:
            cp.wait()
    pltpu.sync_copy(rows_v, out_hbm.at[wid])


def _sc_scatter_body(ef_hbm, dst_hbm, zero_hbm, part_hbm, idx_v, ef_v, acc, sem):
    c = lax.axis_index("c")
    s = lax.axis_index("s")
    wid = s * NC + c
    pltpu.sync_copy(zero_hbm.at[pl.ds(s * ZR, ZR)], acc.at[pl.ds(s * ZR, ZR)])
    pltpu.sync_copy(dst_hbm.at[wid], idx_v)
    pltpu.sync_copy(ef_hbm.at[wid], ef_v)
    plsc.subcore_barrier()
    for g in range(0, NCH, 8):
        cps = [
            pltpu.async_copy(ef_v.at[g + b], acc.at[idx_v.at[g + b]], sem, add=True)
            for b in range(8)
        ]
        for cp in cps:
            cp.wait()
    plsc.subcore_barrier()
    pltpu.sync_copy(acc.at[pl.ds(s * ZR, ZR)], part_hbm.at[c, pl.ds(s * ZR, ZR)])


_BLK = 2048
_INV_SQRT3 = 1.0 / math.sqrt(3.0)


def _tc_edge_body(scal_ref, attr_ref, x_ref, w1_ref, w2_ref, out_ref):
    h = jnp.maximum(
        jnp.dot(scal_ref[...], w1_ref[...], preferred_element_type=jnp.float32), 0.0
    )
    w = jnp.dot(h, w2_ref[...], preferred_element_type=jnp.float32)
    k_i = jax.lax.broadcasted_iota(jnp.int32, (D, HID), 1) // D
    r_i = jax.lax.broadcasted_iota(jnp.int32, (D, HID), 0)
    rmat = (k_i == r_i).astype(jnp.float32)
    k_o = jax.lax.broadcasted_iota(jnp.int32, (HID, D), 0) % D
    o_o = jax.lax.broadcasted_iota(jnp.int32, (HID, D), 1)
    smat = (k_o == o_o).astype(jnp.float32)
    xr = jnp.dot(x_ref[...], rmat, preferred_element_type=jnp.float32)
    ef = jnp.dot(xr * w, smat, preferred_element_type=jnp.float32)
    out_ref[...] = ef * attr_ref[...]


def _tc_edge(scal, attr, x, w1, w2):
    return pl.pallas_call(
        _tc_edge_body,
        grid=(E_PAD // _BLK,),
        in_specs=[
            pl.BlockSpec((_BLK, 3), lambda i: (i, 0)),
            pl.BlockSpec((_BLK, 1), lambda i: (i, 0)),
            pl.BlockSpec((_BLK, D), lambda i: (i, 0)),
            pl.BlockSpec((3, HID), lambda i: (0, 0)),
            pl.BlockSpec((HID, HID), lambda i: (0, 0)),
        ],
        out_specs=pl.BlockSpec((_BLK, D), lambda i: (i, 0)),
        out_shape=jax.ShapeDtypeStruct((E_PAD, D), jnp.float32),
    )(scal, attr, x, w1, w2)


def _tc_combine_body(p_ref, o_ref):
    o_ref[...] = p_ref[0] + p_ref[1]


def _tc_combine(parts):
    return pl.pallas_call(
        _tc_combine_body,
        grid=(5,),
        in_specs=[pl.BlockSpec((NC, N // 5, D), lambda i: (0, i, 0))],
        out_specs=pl.BlockSpec((N // 5, D), lambda i: (i, 0)),
        out_shape=jax.ShapeDtypeStruct((N, D), jnp.float32),
    )(parts)


@functools.cache
def _sc_kernels():
    mesh = plsc.VectorSubcoreMesh(core_axis_name="c", subcore_axis_name="s")
    gather = pl.kernel(
        _sc_gather_body,
        out_type=jax.ShapeDtypeStruct((NW, NCH, CH, D), jnp.float32),
        mesh=mesh,
        scratch_types=[
            pltpu.VMEM((NCH, CH), jnp.int32),
            pltpu.VMEM((NCH, CH, D), jnp.float32),
            pltpu.SemaphoreType.DMA,
        ],
        compiler_params=pltpu.CompilerParams(use_tc_tiling_on_sc=False),
    )
    scatter = pl.kernel(
        _sc_scatter_body,
        out_type=jax.ShapeDtypeStruct((NC, N, D), jnp.float32),
        mesh=mesh,
        scratch_types=[
            pltpu.VMEM((NCH, CH), jnp.int32),
            pltpu.VMEM((NCH, CH, D), jnp.float32),
            pltpu.VMEM_SHARED((N, D), jnp.float32),
            pltpu.SemaphoreType.DMA,
        ],
        compiler_params=pltpu.CompilerParams(use_tc_tiling_on_sc=False),
    )
    return gather, scatter


def kernel(node_features, edge_src, edge_dst, edge_attr, edge_scalars, W1, W2):
    _sc_gather, _sc_scatter = _sc_kernels()
    pad = E_PAD - E
    src = jnp.pad(edge_src, (0, pad)).reshape(NW, NCH, CH)
    dst = jnp.pad(edge_dst, (0, pad)).reshape(NW, NCH, CH)
    attr = jnp.pad(edge_attr, ((0, pad), (0, 0)))
    scal = jnp.pad(edge_scalars, ((0, pad), (0, 0)))

    w1s = W1 * _INV_SQRT3
    w2s = W2 * (1.0 / 256.0)
    x = _sc_gather(node_features, src).reshape(E_PAD, D)
    ef = _tc_edge(scal, attr, x, w1s, w2s).reshape(NW, NCH, CH, D)
    zeros = jnp.zeros((N, D), jnp.float32)
    parts = _sc_scatter(ef, dst, zeros)
    return _tc_combine(parts)


# speedup vs baseline: 5.7430x; 2.0604x over previous
import functools
import math

import jax
import jax.numpy as jnp
from jax import lax
from jax.experimental import pallas as pl
from jax.experimental.pallas import tpu as pltpu
from jax.experimental.pallas import tpu_sc as plsc

N = 10000
E = 160000
D = 16
HID = 256

NC = 2
NS = 16
NW = NC * NS
CH = 128
NCH = 40
PER_W = NCH * CH
E_PAD = NW * PER_W
ZR = N // NS


def _sc_gather_body(nf_hbm, src_hbm, out_hbm, idx_v, rows_v, sem):
    c = lax.axis_index("c")
    s = lax.axis_index("s")
    wid = s * NC + c
    pltpu.sync_copy(src_hbm.at[wid], idx_v)
    for g in range(0, NCH, 8):
        cps = [
            pltpu.async_copy(nf_hbm.at[idx_v.at[g + b]], rows_v.at[g + b], sem)
            for b in range(8)
        ]
        for cp in cps:
            cp.wait()
    pltpu.sync_copy(rows_v, out_hbm.at[wid])


def _sc_scatter_body(ef_hbm, dst_hbm, zero_hbm, part_hbm, idx_v, ef_v, acc, sem):
    c = lax.axis_index("c")
    s = lax.axis_index("s")
    wid = s * NC + c
    pltpu.sync_copy(zero_hbm.at[pl.ds(s * ZR, ZR)], acc.at[pl.ds(s * ZR, ZR)])
    pltpu.sync_copy(dst_hbm.at[wid], idx_v)
    pltpu.sync_copy(ef_hbm.at[wid], ef_v)
    plsc.subcore_barrier()
    for g in range(0, NCH, 8):
        cps = [
            pltpu.async_copy(ef_v.at[g + b], acc.at[idx_v.at[g + b]], sem, add=True)
            for b in range(8)
        ]
        for cp in cps:
            cp.wait()
    plsc.subcore_barrier()
    pltpu.sync_copy(acc.at[pl.ds(s * ZR, ZR)], part_hbm.at[c, pl.ds(s * ZR, ZR)])


_BLK = 2048
_INV_SQRT3 = 1.0 / math.sqrt(3.0)


def _tc_edge_body(sat_ref, x_ref, w1_ref, w2_ref, rmat_ref, smat_ref, out_ref):
    sal = jnp.transpose(sat_ref[...])
    h = jnp.maximum(
        jnp.dot(sal, w1_ref[...], preferred_element_type=jnp.float32), 0.0
    )
    w = jnp.dot(h, w2_ref[...], preferred_element_type=jnp.float32)
    w = w * sal[:, 3:4]
    w3 = w.reshape(_BLK // 8, 8, HID)
    xp = x_ref[...]
    efs = []
    for m in range(8):
        xs = xp[:, D * m : D * (m + 1)]
        xr = jnp.dot(xs, rmat_ref[...], preferred_element_type=jnp.float32)
        efs.append(
            jnp.dot(xr * w3[:, m, :], smat_ref[...], preferred_element_type=jnp.float32)
        )
    out_ref[...] = jnp.concatenate(efs, axis=1)


def _tc_edge(sat, x_packed, w1, w2, rmat, smat):
    return pl.pallas_call(
        _tc_edge_body,
        grid=(E_PAD // _BLK,),
        in_specs=[
            pl.BlockSpec((4, _BLK), lambda i: (0, i)),
            pl.BlockSpec((_BLK // 8, 128), lambda i: (i, 0)),
            pl.BlockSpec((4, HID), lambda i: (0, 0)),
            pl.BlockSpec((HID, HID), lambda i: (0, 0)),
            pl.BlockSpec((D, HID), lambda i: (0, 0)),
            pl.BlockSpec((HID, D), lambda i: (0, 0)),
        ],
        out_specs=pl.BlockSpec((_BLK // 8, 128), lambda i: (i, 0)),
        out_shape=jax.ShapeDtypeStruct((E_PAD // 8, 128), jnp.float32),
    )(sat, x_packed, w1, w2, rmat, smat)


def _tc_combine_body(p_ref, o_ref):
    o_ref[...] = p_ref[0] + p_ref[1]


def _tc_combine(parts):
    return pl.pallas_call(
        _tc_combine_body,
        grid=(5,),
        in_specs=[pl.BlockSpec((NC, N // 5, D), lambda i: (0, i, 0))],
        out_specs=pl.BlockSpec((N // 5, D), lambda i: (i, 0)),
        out_shape=jax.ShapeDtypeStruct((N, D), jnp.float32),
    )(parts)


@functools.cache
def _sc_kernels():
    mesh = plsc.VectorSubcoreMesh(core_axis_name="c", subcore_axis_name="s")
    gather = pl.kernel(
        _sc_gather_body,
        out_type=jax.ShapeDtypeStruct((NW, NCH, CH, D), jnp.float32),
        mesh=mesh,
        scratch_types=[
            pltpu.VMEM((NCH, CH), jnp.int32),
            pltpu.VMEM((NCH, CH, D), jnp.float32),
            pltpu.SemaphoreType.DMA,
        ],
        compiler_params=pltpu.CompilerParams(use_tc_tiling_on_sc=False),
    )
    scatter = pl.kernel(
        _sc_scatter_body,
        out_type=jax.ShapeDtypeStruct((NC, N, D), jnp.float32),
        mesh=mesh,
        scratch_types=[
            pltpu.VMEM((NCH, CH), jnp.int32),
            pltpu.VMEM((NCH, CH, D), jnp.float32),
            pltpu.VMEM_SHARED((N, D), jnp.float32),
            pltpu.SemaphoreType.DMA,
        ],
        compiler_params=pltpu.CompilerParams(use_tc_tiling_on_sc=False),
    )
    return gather, scatter


def kernel(node_features, edge_src, edge_dst, edge_attr, edge_scalars, W1, W2):
    _sc_gather, _sc_scatter = _sc_kernels()
    pad = E_PAD - E
    src = jnp.pad(edge_src, (0, pad)).reshape(NW, NCH, CH)
    dst = jnp.pad(edge_dst, (0, pad)).reshape(NW, NCH, CH)
    sat = jnp.pad(
        jnp.concatenate([edge_scalars.T, edge_attr.T], axis=0), ((0, 0), (0, pad))
    )

    w1s = jnp.pad(W1 * _INV_SQRT3, ((0, 1), (0, 0)))
    w2s = W2 * (1.0 / 256.0)
    i16 = jnp.arange(D, dtype=jnp.int32)
    k256 = jnp.arange(HID, dtype=jnp.int32)
    rmat = (k256[None, :] // D == i16[:, None]).astype(jnp.float32)
    smat = (k256[:, None] % D == i16[None, :]).astype(jnp.float32)

    x_packed = _sc_gather(node_features, src).reshape(E_PAD // 8, 128)
    ef = _tc_edge(sat, x_packed, w1s, w2s, rmat, smat).reshape(NW, NCH, CH, D)
    zeros = jnp.zeros((N, D), jnp.float32)
    parts = _sc_scatter(ef, dst, zeros)
    return _tc_combine(parts)


# speedup vs baseline: 5.7782x; 1.0061x over previous
import functools
import math

import jax
import jax.numpy as jnp
from jax import lax
from jax.experimental import pallas as pl
from jax.experimental.pallas import tpu as pltpu
from jax.experimental.pallas import tpu_sc as plsc

N = 10000
E = 160000
D = 16
HID = 256

NC = 2
NS = 16
NW = NC * NS
CH = 128
NCH = 40
PER_W = NCH * CH
E_PAD = NW * PER_W
ZR = N // NS


def _sc_gather_body(nf_hbm, src_hbm, out_hbm, idx_v, rows_v, sem):
    c = lax.axis_index("c")
    s = lax.axis_index("s")
    wid = s * NC + c
    pltpu.sync_copy(src_hbm.at[wid], idx_v)
    for g in range(0, NCH, 8):
        cps = [
            pltpu.async_copy(nf_hbm.at[idx_v.at[g + b]], rows_v.at[g + b], sem)
            for b in range(8)
        ]
        for cp in cps:
            cp.wait()
    pltpu.sync_copy(rows_v, out_hbm.at[wid])


def _sc_scatter_body(ef_hbm, dst_hbm, zero_hbm, part_hbm, idx_v, ef_v, acc, sem):
    c = lax.axis_index("c")
    s = lax.axis_index("s")
    wid = s * NC + c
    pltpu.sync_copy(zero_hbm.at[pl.ds(s * ZR, ZR)], acc.at[pl.ds(s * ZR, ZR)])
    pltpu.sync_copy(dst_hbm.at[wid], idx_v)
    pltpu.sync_copy(ef_hbm.at[wid], ef_v)
    plsc.subcore_barrier()
    for g in range(0, NCH, 8):
        cps = [
            pltpu.async_copy(ef_v.at[g + b], acc.at[idx_v.at[g + b]], sem, add=True)
            for b in range(8)
        ]
        for cp in cps:
            cp.wait()
    plsc.subcore_barrier()
    pltpu.sync_copy(acc.at[pl.ds(s * ZR, ZR)], part_hbm.at[c, pl.ds(s * ZR, ZR)])


_BLK = 2048
_INV_SQRT3 = 1.0 / math.sqrt(3.0)


def _tc_edge_body(sat_ref, x_ref, w1_ref, w2_ref, rmat_ref, smat_ref, out_ref):
    sal = jnp.transpose(sat_ref[...])
    h = jnp.maximum(
        jnp.dot(sal, w1_ref[...], preferred_element_type=jnp.float32), 0.0
    )
    w = jnp.dot(
        h.astype(jnp.bfloat16), w2_ref[...], preferred_element_type=jnp.float32
    )
    w = w * sal[:, 3:4]
    w3 = w.reshape(_BLK // 8, 8, HID)
    xp = x_ref[...]
    efs = []
    for m in range(8):
        xs = xp[:, D * m : D * (m + 1)]
        xr = jnp.dot(xs, rmat_ref[...], preferred_element_type=jnp.float32)
        efs.append(
            jnp.dot(xr * w3[:, m, :], smat_ref[...], preferred_element_type=jnp.float32)
        )
    out_ref[...] = jnp.concatenate(efs, axis=1)


def _tc_edge(sat, x_packed, w1, w2, rmat, smat):
    return pl.pallas_call(
        _tc_edge_body,
        grid=(E_PAD // _BLK,),
        in_specs=[
            pl.BlockSpec((4, _BLK), lambda i: (0, i)),
            pl.BlockSpec((_BLK // 8, 128), lambda i: (i, 0)),
            pl.BlockSpec((4, HID), lambda i: (0, 0)),
            pl.BlockSpec((HID, HID), lambda i: (0, 0), memory_space=pltpu.ANY)
            if False
            else pl.BlockSpec((HID, HID), lambda i: (0, 0)),
            pl.BlockSpec((D, HID), lambda i: (0, 0)),
            pl.BlockSpec((HID, D), lambda i: (0, 0)),
        ],
        out_specs=pl.BlockSpec((_BLK // 8, 128), lambda i: (i, 0)),
        out_shape=jax.ShapeDtypeStruct((E_PAD // 8, 128), jnp.float32),
    )(sat, x_packed, w1, w2, rmat, smat)


def _tc_combine_body(p_ref, o_ref):
    o_ref[...] = p_ref[0] + p_ref[1]


def _tc_combine(parts):
    return pl.pallas_call(
        _tc_combine_body,
        grid=(5,),
        in_specs=[pl.BlockSpec((NC, N // 5, D), lambda i: (0, i, 0))],
        out_specs=pl.BlockSpec((N // 5, D), lambda i: (i, 0)),
        out_shape=jax.ShapeDtypeStruct((N, D), jnp.float32),
    )(parts)


@functools.cache
def _sc_kernels():
    mesh = plsc.VectorSubcoreMesh(core_axis_name="c", subcore_axis_name="s")
    gather = pl.kernel(
        _sc_gather_body,
        out_type=jax.ShapeDtypeStruct((NW, NCH, CH, D), jnp.float32),
        mesh=mesh,
        scratch_types=[
            pltpu.VMEM((NCH, CH), jnp.int32),
            pltpu.VMEM((NCH, CH, D), jnp.float32),
            pltpu.SemaphoreType.DMA,
        ],
        compiler_params=pltpu.CompilerParams(use_tc_tiling_on_sc=False),
    )
    scatter = pl.kernel(
        _sc_scatter_body,
        out_type=jax.ShapeDtypeStruct((NC, N, D), jnp.float32),
        mesh=mesh,
        scratch_types=[
            pltpu.VMEM((NCH, CH), jnp.int32),
            pltpu.VMEM((NCH, CH, D), jnp.float32),
            pltpu.VMEM_SHARED((N, D), jnp.float32),
            pltpu.SemaphoreType.DMA,
        ],
        compiler_params=pltpu.CompilerParams(use_tc_tiling_on_sc=False),
    )
    return gather, scatter


def kernel(node_features, edge_src, edge_dst, edge_attr, edge_scalars, W1, W2):
    _sc_gather, _sc_scatter = _sc_kernels()
    pad = E_PAD - E
    src = jnp.pad(edge_src, (0, pad)).reshape(NW, NCH, CH)
    dst = jnp.pad(edge_dst, (0, pad)).reshape(NW, NCH, CH)
    sat = jnp.pad(
        jnp.concatenate([edge_scalars.T, edge_attr.T], axis=0), ((0, 0), (0, pad))
    )

    w1s = jnp.pad(W1 * _INV_SQRT3, ((0, 1), (0, 0)))
    w2s = W2 * (1.0 / 256.0)
    i16 = jnp.arange(D, dtype=jnp.int32)
    k256 = jnp.arange(HID, dtype=jnp.int32)
    rmat = (k256[None, :] // D == i16[:, None]).astype(jnp.float32)
    smat = (k256[:, None] % D == i16[None, :]).astype(jnp.float32)

    x_packed = _sc_gather(node_features, src).reshape(E_PAD // 8, 128)
    ef = _tc_edge(sat, x_packed, w1s, w2s, rmat, smat).reshape(NW, NCH, CH, D)
    zeros = jnp.zeros((N, D), jnp.float32)
    parts = _sc_scatter(ef, dst, zeros)
    return _tc_combine(parts)
